# Initial kernel scaffold; baseline (speedup 1.0000x reference)
import functools

import jax
import jax.numpy as jnp
from jax import lax
from jax.experimental import pallas as pl
from jax.experimental.pallas import tpu as pltpu
from jax.experimental.pallas import tpu_sc as plsc

NC = 2
NS = 16
NW = NC * NS

B = 4096
L = 50
DIM = 32
NT = 4
N = B * L
CH = 128
NROW = N // CH
ROWS_PER_W = NROW // NW


def _emb_kernel(ids0, ids1, ids2, ids3, t0, t1, t2, t3, out_hbm,
                idx_v, rows_v, sem):
    wid = lax.axis_index("s") * NC + lax.axis_index("c")
    row0 = wid * ROWS_PER_W
    tables = (t0, t1, t2, t3)
    ids = (ids0, ids1, ids2, ids3)
    for t in range(NT):
        pltpu.sync_copy(ids[t].at[pl.ds(row0, ROWS_PER_W)], idx_v)

        def body(j, carry, t=t):
            pltpu.async_copy(tables[t].at[idx_v.at[j]], rows_v, sem).wait()
            pltpu.sync_copy(rows_v,
                            out_hbm.at[pl.ds((row0 + j) * CH, CH), t])
            return carry

        lax.fori_loop(0, ROWS_PER_W, body, 0)


@jax.jit
def _run(ids0, ids1, ids2, ids3, t0, t1, t2, t3):
    mesh = plsc.VectorSubcoreMesh(core_axis_name="c", subcore_axis_name="s",
                                  num_cores=NC, num_subcores=NS)
    k = pl.kernel(
        _emb_kernel,
        out_type=jax.ShapeDtypeStruct((N, NT, DIM), jnp.float32),
        mesh=mesh,
        scratch_types=[
            pltpu.VMEM((ROWS_PER_W, CH), jnp.int32),
            pltpu.VMEM((CH, DIM), jnp.float32),
            pltpu.SemaphoreType.DMA,
        ],
    )
    return k(ids0, ids1, ids2, ids3, t0, t1, t2, t3)


def kernel(ids_0, ids_1, ids_2, ids_3, table_0, table_1, table_2, table_3):
    ids = [i.astype(jnp.int32).reshape(NROW, CH)
           for i in (ids_0, ids_1, ids_2, ids_3)]
    out = _run(*ids, table_0, table_1, table_2, table_3)
    return out.reshape(B, L, NT * DIM)


# speedup vs baseline: 1.8962x; 1.8962x over previous
import functools

import jax
import jax.numpy as jnp
from jax import lax
from jax.experimental import pallas as pl
from jax.experimental.pallas import tpu as pltpu
from jax.experimental.pallas import tpu_sc as plsc

NC = 2
NS = 16
NW = NC * NS

B = 4096
L = 50
DIM = 32
NT = 4
N = B * L
CH = 128
NROW = N // CH
ROWS_PER_W = NROW // NW


def _emb_kernel(ids0, ids1, ids2, ids3, t0, t1, t2, t3, out_hbm,
                idx_v, rows_v, sem):
    wid = lax.axis_index("s") * NC + lax.axis_index("c")
    row0 = wid * ROWS_PER_W
    tables = (t0, t1, t2, t3)
    ids = (ids0, ids1, ids2, ids3)
    for t in range(NT):
        pltpu.sync_copy(ids[t].at[pl.ds(row0, ROWS_PER_W)], idx_v)

        def body(j, carry, t=t):
            pltpu.async_copy(tables[t].at[idx_v.at[j]], rows_v, sem).wait()
            pltpu.sync_copy(rows_v,
                            out_hbm.at[pl.ds((row0 + j) * CH, CH), t])
            return carry

        lax.fori_loop(0, ROWS_PER_W, body, 0)


@jax.jit
def _run(ids0, ids1, ids2, ids3, t0, t1, t2, t3):
    mesh = plsc.VectorSubcoreMesh(core_axis_name="c", subcore_axis_name="s",
                                  num_cores=NC, num_subcores=NS)
    k = pl.kernel(
        _emb_kernel,
        out_type=jax.ShapeDtypeStruct((N, NT, DIM), jnp.float32),
        mesh=mesh,
        scratch_types=[
            pltpu.VMEM((ROWS_PER_W, CH), jnp.int32),
            pltpu.VMEM((CH, DIM), jnp.float32),
            pltpu.SemaphoreType.DMA,
        ],
        compiler_params=pltpu.CompilerParams(use_tc_tiling_on_sc=False),
    )
    return k(ids0, ids1, ids2, ids3, t0, t1, t2, t3)


def kernel(ids_0, ids_1, ids_2, ids_3, table_0, table_1, table_2, table_3):
    ids = [i.astype(jnp.int32).reshape(NROW, CH)
           for i in (ids_0, ids_1, ids_2, ids_3)]
    out = _run(*ids, table_0, table_1, table_2, table_3)
    return out.reshape(B, L, NT * DIM)


# speedup vs baseline: 2.2237x; 1.1727x over previous
import functools

import jax
import jax.numpy as jnp
from jax import lax
from jax.experimental import pallas as pl
from jax.experimental.pallas import tpu as pltpu
from jax.experimental.pallas import tpu_sc as plsc

NC = 2
NS = 16
NW = NC * NS

B = 4096
L = 50
DIM = 32
NT = 4
CB = B // NW


def _emb_kernel(ids0, ids1, ids2, ids3, t0, t1, t2, t3, out_hbm,
                idx_v, rows_v, sem):
    wid = lax.axis_index("s") * NC + lax.axis_index("c")
    b0 = wid * CB
    tables = (t0, t1, t2, t3)
    ids = (ids0, ids1, ids2, ids3)
    for t in range(NT):
        pltpu.sync_copy(ids[t].at[:, pl.ds(b0, CB)], idx_v)

        def body(l, carry, t=t):
            pltpu.async_copy(tables[t].at[idx_v.at[l]], rows_v, sem).wait()
            pltpu.sync_copy(
                rows_v,
                out_hbm.at[l, pl.ds(b0, CB), pl.ds(t * DIM, DIM)])
            return carry

        lax.fori_loop(0, L, body, 0)


@jax.jit
def _run(ids0, ids1, ids2, ids3, t0, t1, t2, t3):
    mesh = plsc.VectorSubcoreMesh(core_axis_name="c", subcore_axis_name="s",
                                  num_cores=NC, num_subcores=NS)
    k = pl.kernel(
        _emb_kernel,
        out_type=jax.ShapeDtypeStruct((L, B, NT * DIM), jnp.float32),
        mesh=mesh,
        scratch_types=[
            pltpu.VMEM((L, CB), jnp.int32),
            pltpu.VMEM((CB, DIM), jnp.float32),
            pltpu.SemaphoreType.DMA,
        ],
        compiler_params=pltpu.CompilerParams(use_tc_tiling_on_sc=False),
    )
    return k(ids0, ids1, ids2, ids3, t0, t1, t2, t3)


def kernel(ids_0, ids_1, ids_2, ids_3, table_0, table_1, table_2, table_3):
    ids = [i.astype(jnp.int32).T for i in (ids_0, ids_1, ids_2, ids_3)]
    out = _run(*ids, table_0, table_1, table_2, table_3)
    return out.transpose(1, 0, 2)
